# baseline (speedup 1.0000x reference)
import jax
import jax.numpy as jnp
import numpy as np
from jax.experimental import pallas as pl
from jax.experimental.pallas import tpu as pltpu

_N = 2048
_P = 5
_LANES = 128
_COLS = _P * _LANES
_ROWS = _N * _N // _LANES
_BR = 1024


def _seg_matrix() -> np.ndarray:
    s = np.zeros((_COLS, _LANES), dtype=np.float32)
    for i in range(_COLS):
        s[i, i // _P] = 1.0
    return s


def _body(tab_ref, p_ref, s_ref, o_ref):
    blk = p_ref[...]
    mask = (blk != -1).astype(jnp.bfloat16)
    counts = jax.lax.dot_general(
        mask, s_ref[...],
        dimension_numbers=(((1,), (0,)), ((), ())),
        preferred_element_type=jnp.float32,
    )
    out = jnp.where(counts == 0, jnp.float32(0.0), tab_ref[0])
    out = jnp.where(counts == 2, tab_ref[1], out)
    out = jnp.where(counts == 3, tab_ref[2], out)
    out = jnp.where(counts == 4, tab_ref[3], out)
    out = jnp.where(counts == 5, tab_ref[4], out)
    o_ref[...] = out


@jax.jit
def kernel(x, paths, b):
    del x
    flat = paths.reshape(_ROWS, _COLS)
    seg = jnp.asarray(_seg_matrix(), dtype=jnp.bfloat16)
    grid = (_ROWS // _BR,)
    out = pl.pallas_call(
        _body,
        grid=grid,
        in_specs=[
            pl.BlockSpec(memory_space=pltpu.SMEM),
            pl.BlockSpec((_BR, _COLS), lambda i: (i, 0)),
            pl.BlockSpec((_COLS, _LANES), lambda i: (0, 0)),
        ],
        out_specs=pl.BlockSpec((_BR, _LANES), lambda i: (i, 0)),
        out_shape=jax.ShapeDtypeStruct((_ROWS, _LANES), jnp.float32),
        compiler_params=pltpu.CompilerParams(
            dimension_semantics=("arbitrary",),
        ),
    )(b, flat, seg)
    return out.reshape(_N, _N)


# speedup vs baseline: 48.6483x; 48.6483x over previous
import jax
import jax.numpy as jnp
from jax.experimental import pallas as pl
from jax.experimental.pallas import tpu as pltpu

_N = 2048
_P = 5
_BR = 256


def _body(tab_ref, p_ref, o_ref):
    inv = jax.lax.shift_right_logical(p_ref[0], 31)
    for k in range(1, _P):
        inv = inv + jax.lax.shift_right_logical(p_ref[k], 31)
    counts = _P - inv
    out = jnp.where(counts == 0, jnp.float32(0.0), tab_ref[0])
    out = jnp.where(counts == 2, tab_ref[1], out)
    out = jnp.where(counts == 3, tab_ref[2], out)
    out = jnp.where(counts == 4, tab_ref[3], out)
    out = jnp.where(counts == 5, tab_ref[4], out)
    o_ref[...] = out


@jax.jit
def kernel(x, paths, b):
    del x
    planes = jnp.moveaxis(paths, -1, 0)
    grid = (_N // _BR,)
    return pl.pallas_call(
        _body,
        grid=grid,
        in_specs=[
            pl.BlockSpec(memory_space=pltpu.SMEM),
            pl.BlockSpec((_P, _BR, _N), lambda i: (0, i, 0)),
        ],
        out_specs=pl.BlockSpec((_BR, _N), lambda i: (i, 0)),
        out_shape=jax.ShapeDtypeStruct((_N, _N), jnp.float32),
        compiler_params=pltpu.CompilerParams(
            dimension_semantics=("arbitrary",),
        ),
    )(b, planes)
